# baseline (speedup 1.0000x reference)
import functools

import jax
import jax.numpy as jnp
from jax import lax
from jax.experimental import pallas as pl
from jax.experimental.pallas import tpu as pltpu
from jax.experimental.pallas import tpu_sc as plsc

SIZE_OUT = 128
N_INPUTS = 64
BATCH = 16384
L = 16
NUM_CORES = 2
NUM_SUBCORES = 16
NW = NUM_CORES * NUM_SUBCORES
ROWS = BATCH // NW

_mesh = plsc.VectorSubcoreMesh(core_axis_name="c", subcore_axis_name="s")


@functools.partial(
    pl.kernel,
    mesh=_mesh,
    compiler_params=pltpu.CompilerParams(needs_layout_passes=False),
    out_type=jax.ShapeDtypeStruct((BATCH * SIZE_OUT,), jnp.float32),
    scratch_types=[
        pltpu.VMEM((ROWS * N_INPUTS,), jnp.float32),
        pltpu.VMEM((ROWS * SIZE_OUT,), jnp.float32),
        pltpu.VMEM((N_INPUTS,), jnp.float32),
        pltpu.VMEM((N_INPUTS,), jnp.int32),
    ],
)
def _sc_scatter(x_hbm, w_hbm, ord_hbm, out_hbm, x_v, o_v, w_v, ord_v):
    wid = lax.axis_index("s") * NUM_CORES + lax.axis_index("c")
    base = wid * ROWS

    pltpu.sync_copy(w_hbm, w_v)
    pltpu.sync_copy(ord_hbm, ord_v)
    pltpu.sync_copy(x_hbm.at[pl.ds(base * N_INPUTS, ROWS * N_INPUTS)], x_v)

    n_in_chunks = N_INPUTS // L
    n_out_chunks = SIZE_OUT // L
    w_c = [w_v[pl.ds(c * L, L)] for c in range(n_in_chunks)]
    ord_c = [ord_v[pl.ds(c * L, L)] for c in range(n_in_chunks)]
    zeros = jnp.zeros((L,), jnp.float32)

    def row_body(r, carry):
        o_base = r * SIZE_OUT
        for k in range(n_out_chunks):
            o_v[pl.ds(o_base + k * L, L)] = zeros
        for c in range(n_in_chunks):
            val = x_v[pl.ds(r * N_INPUTS + c * L, L)] * w_c[c]
            plsc.store_scatter(o_v, [ord_c[c] + o_base], val)
        return carry

    lax.fori_loop(0, ROWS, row_body, 0)

    pltpu.sync_copy(o_v, out_hbm.at[pl.ds(base * SIZE_OUT, ROWS * SIZE_OUT)])


def kernel(X_in, weights, input_node_order):
    out_flat = _sc_scatter(
        X_in.reshape(-1),
        weights.astype(jnp.float32),
        input_node_order.astype(jnp.int32),
    )
    return out_flat.reshape(BATCH, SIZE_OUT)


# speedup vs baseline: 1.1242x; 1.1242x over previous
import functools

import jax
import jax.numpy as jnp
from jax import lax
from jax.experimental import pallas as pl
from jax.experimental.pallas import tpu as pltpu
from jax.experimental.pallas import tpu_sc as plsc

SIZE_OUT = 128
N_INPUTS = 64
BATCH = 16384
L = 16
NUM_CORES = 2
NUM_SUBCORES = 16
NW = NUM_CORES * NUM_SUBCORES
ROWS = BATCH // NW
BLK = 256
NBLK = ROWS // BLK

_mesh = plsc.VectorSubcoreMesh(core_axis_name="c", subcore_axis_name="s")


@functools.partial(
    pl.kernel,
    mesh=_mesh,
    compiler_params=pltpu.CompilerParams(needs_layout_passes=False),
    out_type=jax.ShapeDtypeStruct((BATCH, SIZE_OUT), jnp.float32),
    scratch_types=[
        pltpu.VMEM((BLK, N_INPUTS), jnp.float32),
        pltpu.VMEM((BLK, SIZE_OUT), jnp.float32),
        pltpu.VMEM((N_INPUTS,), jnp.float32),
        pltpu.VMEM((N_INPUTS,), jnp.int32),
    ],
)
def _sc_scatter(x_hbm, w_hbm, ord_hbm, out_hbm, x_v, o_v, w_v, ord_v):
    wid = lax.axis_index("s") * NUM_CORES + lax.axis_index("c")
    base = wid * ROWS

    pltpu.sync_copy(w_hbm, w_v)
    pltpu.sync_copy(ord_hbm, ord_v)

    n_in_chunks = N_INPUTS // L
    n_out_chunks = SIZE_OUT // L
    w_c = [w_v[pl.ds(c * L, L)] for c in range(n_in_chunks)]
    ord_c = [ord_v[pl.ds(c * L, L)] for c in range(n_in_chunks)]
    zeros = jnp.zeros((L,), jnp.float32)

    def row_body(r, carry):
        for k in range(n_out_chunks):
            o_v[r, pl.ds(k * L, L)] = zeros
        row_vec = jnp.full((L,), r, jnp.int32)
        for c in range(n_in_chunks):
            val = x_v[r, pl.ds(c * L, L)] * w_c[c]
            plsc.store_scatter(o_v, [row_vec, ord_c[c]], val)
        return carry

    for b in range(NBLK):
        blk_base = base + b * BLK
        pltpu.sync_copy(x_hbm.at[pl.ds(blk_base, BLK)], x_v)
        lax.fori_loop(0, BLK, row_body, 0)
        pltpu.sync_copy(o_v, out_hbm.at[pl.ds(blk_base, BLK)])


def kernel(X_in, weights, input_node_order):
    return _sc_scatter(
        X_in,
        weights.astype(jnp.float32),
        input_node_order.astype(jnp.int32),
    )


# speedup vs baseline: 1.3012x; 1.1574x over previous
import functools

import jax
import jax.numpy as jnp
from jax import lax
from jax.experimental import pallas as pl
from jax.experimental.pallas import tpu as pltpu
from jax.experimental.pallas import tpu_sc as plsc

SIZE_OUT = 128
N_INPUTS = 64
BATCH = 16384
L = 16
NUM_CORES = 2
NUM_SUBCORES = 16
NW = NUM_CORES * NUM_SUBCORES
ROWS = BATCH // NW
BLK = 128
NBLK = ROWS // BLK
UNROLL = 8

_mesh = plsc.VectorSubcoreMesh(core_axis_name="c", subcore_axis_name="s")


@functools.partial(
    pl.kernel,
    mesh=_mesh,
    compiler_params=pltpu.CompilerParams(needs_layout_passes=False),
    out_type=jax.ShapeDtypeStruct((BATCH, SIZE_OUT), jnp.float32),
    scratch_types=[
        pltpu.VMEM((BLK, N_INPUTS), jnp.float32),
        pltpu.VMEM((BLK, N_INPUTS), jnp.float32),
        pltpu.VMEM((BLK, SIZE_OUT), jnp.float32),
        pltpu.VMEM((BLK, SIZE_OUT), jnp.float32),
        pltpu.VMEM((N_INPUTS,), jnp.float32),
        pltpu.VMEM((N_INPUTS,), jnp.int32),
        pltpu.SemaphoreType.DMA,
        pltpu.SemaphoreType.DMA,
        pltpu.SemaphoreType.DMA,
        pltpu.SemaphoreType.DMA,
    ],
)
def _sc_scatter(
    x_hbm, w_hbm, ord_hbm, out_hbm,
    x0, x1, o0, o1, w_v, ord_v, si0, si1, so0, so1,
):
    wid = lax.axis_index("s") * NUM_CORES + lax.axis_index("c")
    base = wid * ROWS

    xb, ob = [x0, x1], [o0, o1]
    sin, sout = [si0, si1], [so0, so1]

    in_dma = [None] * NBLK
    out_dma = [None] * NBLK
    in_dma[0] = pltpu.async_copy(x_hbm.at[pl.ds(base, BLK)], xb[0], sin[0])
    pltpu.sync_copy(w_hbm, w_v)
    pltpu.sync_copy(ord_hbm, ord_v)

    n_in_chunks = N_INPUTS // L
    n_out_chunks = SIZE_OUT // L
    w_c = [w_v[pl.ds(c * L, L)] for c in range(n_in_chunks)]
    ord_c = [ord_v[pl.ds(c * L, L)] for c in range(n_in_chunks)]
    zeros = jnp.zeros((L,), jnp.float32)

    def zero_body(r, carry):
        for k in range(n_out_chunks):
            o0[r, pl.ds(k * L, L)] = zeros
            o1[r, pl.ds(k * L, L)] = zeros
        return carry

    lax.fori_loop(0, BLK, zero_body, 0)

    def make_row_loop(x_ref, o_ref):
        def row_body(t, carry):
            for u in range(UNROLL):
                r = t * UNROLL + u
                row_vec = jnp.full((L,), r, jnp.int32)
                for c in range(n_in_chunks):
                    val = x_ref[r, pl.ds(c * L, L)] * w_c[c]
                    plsc.store_scatter(o_ref, [row_vec, ord_c[c]], val)
            return carry

        return row_body

    for b in range(NBLK):
        i = b & 1
        if b + 1 < NBLK:
            in_dma[b + 1] = pltpu.async_copy(
                x_hbm.at[pl.ds(base + (b + 1) * BLK, BLK)], xb[(b + 1) & 1],
                sin[(b + 1) & 1],
            )
        in_dma[b].wait()
        if b >= 2:
            out_dma[b - 2].wait()
        lax.fori_loop(0, BLK // UNROLL, make_row_loop(xb[i], ob[i]), 0)
        out_dma[b] = pltpu.async_copy(
            ob[i], out_hbm.at[pl.ds(base + b * BLK, BLK)], sout[i]
        )

    out_dma[NBLK - 2].wait()
    out_dma[NBLK - 1].wait()


def kernel(X_in, weights, input_node_order):
    return _sc_scatter(
        X_in,
        weights.astype(jnp.float32),
        input_node_order.astype(jnp.int32),
    )


# speedup vs baseline: 1.3027x; 1.0012x over previous
import functools

import jax
import jax.numpy as jnp
from jax import lax
from jax.experimental import pallas as pl
from jax.experimental.pallas import tpu as pltpu
from jax.experimental.pallas import tpu_sc as plsc

SIZE_OUT = 128
N_INPUTS = 64
BATCH = 16384
L = 16
NUM_CORES = 2
NUM_SUBCORES = 16
NW = NUM_CORES * NUM_SUBCORES
ROWS = BATCH // NW
BLK = 128
NBLK = ROWS // BLK
UNROLL = 8

_mesh = plsc.VectorSubcoreMesh(core_axis_name="c", subcore_axis_name="s")


@functools.partial(
    pl.kernel,
    mesh=_mesh,
    compiler_params=pltpu.CompilerParams(
        needs_layout_passes=False, use_tc_tiling_on_sc=True
    ),
    out_type=jax.ShapeDtypeStruct((BATCH, SIZE_OUT), jnp.float32),
    scratch_types=[
        pltpu.VMEM((BLK, N_INPUTS), jnp.float32),
        pltpu.VMEM((BLK, N_INPUTS), jnp.float32),
        pltpu.VMEM((BLK, SIZE_OUT), jnp.float32),
        pltpu.VMEM((BLK, SIZE_OUT), jnp.float32),
        pltpu.VMEM((N_INPUTS,), jnp.float32),
        pltpu.VMEM((N_INPUTS,), jnp.int32),
        pltpu.SemaphoreType.DMA,
        pltpu.SemaphoreType.DMA,
        pltpu.SemaphoreType.DMA,
        pltpu.SemaphoreType.DMA,
    ],
)
def _sc_scatter(
    x_hbm, w_hbm, ord_hbm, out_hbm,
    x0, x1, o0, o1, w_v, ord_v, si0, si1, so0, so1,
):
    wid = lax.axis_index("s") * NUM_CORES + lax.axis_index("c")
    base = wid * ROWS

    xb, ob = [x0, x1], [o0, o1]
    sin, sout = [si0, si1], [so0, so1]

    in_dma = [None] * NBLK
    out_dma = [None] * NBLK
    in_dma[0] = pltpu.async_copy(x_hbm.at[pl.ds(base, BLK)], xb[0], sin[0])
    pltpu.sync_copy(w_hbm, w_v)
    pltpu.sync_copy(ord_hbm, ord_v)

    n_in_chunks = N_INPUTS // L
    n_out_chunks = SIZE_OUT // L
    w_c = [w_v[pl.ds(c * L, L)] for c in range(n_in_chunks)]
    ord_c = [ord_v[pl.ds(c * L, L)] for c in range(n_in_chunks)]
    zeros = jnp.zeros((L,), jnp.float32)

    def zero_body(r, carry):
        for k in range(n_out_chunks):
            o0[r, pl.ds(k * L, L)] = zeros
            o1[r, pl.ds(k * L, L)] = zeros
        return carry

    lax.fori_loop(0, BLK, zero_body, 0)

    def make_row_loop(x_ref, o_ref):
        def row_body(t, carry):
            for u in range(UNROLL):
                r = t * UNROLL + u
                row_vec = jnp.full((L,), r, jnp.int32)
                for c in range(n_in_chunks):
                    val = x_ref[r, pl.ds(c * L, L)] * w_c[c]
                    plsc.store_scatter(o_ref, [row_vec, ord_c[c]], val)
            return carry

        return row_body

    for b in range(NBLK):
        i = b & 1
        if b + 1 < NBLK:
            in_dma[b + 1] = pltpu.async_copy(
                x_hbm.at[pl.ds(base + (b + 1) * BLK, BLK)], xb[(b + 1) & 1],
                sin[(b + 1) & 1],
            )
        in_dma[b].wait()
        if b >= 2:
            out_dma[b - 2].wait()
        lax.fori_loop(0, BLK // UNROLL, make_row_loop(xb[i], ob[i]), 0)
        out_dma[b] = pltpu.async_copy(
            ob[i], out_hbm.at[pl.ds(base + b * BLK, BLK)], sout[i]
        )

    out_dma[NBLK - 2].wait()
    out_dma[NBLK - 1].wait()


def kernel(X_in, weights, input_node_order):
    return _sc_scatter(
        X_in,
        weights.astype(jnp.float32),
        input_node_order.astype(jnp.int32),
    )
